# baseline (speedup 1.0000x reference)
import jax
import jax.numpy as jnp
from jax import lax
from jax.experimental import pallas as pl
from jax.experimental.pallas import tpu as pltpu
from jax.experimental.pallas import tpu_sc as plsc

B, S = 4, 4096
D_MODEL = 1024
N_TOK = B * S

_info = plsc.get_sparse_core_info()
NC, NS = _info.num_cores, _info.num_subcores
NW = NC * NS
ROWS_PER_W = N_TOK // NW
W_PER_ROW = S // ROWS_PER_W
CHUNK = 16
N_CHUNKS = ROWS_PER_W // CHUNK
NBUF = 7


def _emb_kernel(table_hbm, idx_hbm, out_hbm, idx_v, *rest):
    bufs = rest[:NBUF]
    gsems = rest[NBUF:2 * NBUF]
    osems = rest[2 * NBUF:3 * NBUF]
    wid = lax.axis_index("s") * NC + lax.axis_index("c")
    pltpu.sync_copy(
        idx_hbm.at[wid // W_PER_ROW,
                   pl.ds((wid % W_PER_ROW) * ROWS_PER_W, ROWS_PER_W)],
        idx_v)
    base = wid * ROWS_PER_W

    def gather(g):
        b = g % NBUF
        return pltpu.async_copy(
            table_hbm.at[idx_v.at[pl.ds(g * CHUNK, CHUNK)]], bufs[b], gsems[b])

    gathers = [None] * N_CHUNKS
    outs = [None] * N_CHUNKS
    for h in range(min(NBUF - 1, N_CHUNKS)):
        gathers[h] = gather(h)
    for g in range(N_CHUNKS):
        b = g % NBUF
        gathers[g].wait()
        outs[g] = pltpu.async_copy(
            bufs[b], out_hbm.at[pl.ds(base + g * CHUNK, CHUNK)], osems[b])
        h = g + NBUF - 1
        if h < N_CHUNKS:
            if h >= NBUF:
                outs[h - NBUF].wait()
            gathers[h] = gather(h)
    for g in range(max(0, N_CHUNKS - NBUF), N_CHUNKS):
        if outs[g] is not None:
            outs[g].wait()


def kernel(tokens, W_E):
    mesh = plsc.VectorSubcoreMesh(core_axis_name="c", subcore_axis_name="s")
    scratch = (
        [pltpu.VMEM((ROWS_PER_W,), jnp.int32)]
        + [pltpu.VMEM((CHUNK, D_MODEL), jnp.float32) for _ in range(NBUF)]
        + [pltpu.SemaphoreType.DMA for _ in range(2 * NBUF)]
    )
    out = pl.kernel(
        _emb_kernel,
        mesh=mesh,
        out_type=jax.ShapeDtypeStruct((N_TOK, D_MODEL), jnp.float32),
        scratch_types=scratch,
        compiler_params=pltpu.CompilerParams(use_tc_tiling_on_sc=False),
    )(W_E, tokens)
    return out.reshape(B, S, D_MODEL)


# speedup vs baseline: 6.2667x; 6.2667x over previous
import jax
import jax.numpy as jnp
from jax import lax
from jax.experimental import pallas as pl
from jax.experimental.pallas import tpu as pltpu
from jax.experimental.pallas import tpu_sc as plsc

B, S = 4, 4096
D_MODEL = 1024
N_TOK = B * S

_info = plsc.get_sparse_core_info()
NC, NS = _info.num_cores, _info.num_subcores
NW = NC * NS
ROWS_PER_W = N_TOK // NW
W_PER_ROW = S // ROWS_PER_W
CHUNK = 16
N_CHUNKS = ROWS_PER_W // CHUNK
NBUF = 7


def _emb_kernel(table_hbm, idx_hbm, out_hbm, idx_v, *rest):
    bufs = rest[:NBUF]
    gsems = rest[NBUF:2 * NBUF]
    osems = rest[2 * NBUF:3 * NBUF]
    wid = lax.axis_index("s") * NC + lax.axis_index("c")
    pltpu.sync_copy(
        idx_hbm.at[wid // W_PER_ROW,
                   pl.ds((wid % W_PER_ROW) * ROWS_PER_W, ROWS_PER_W)],
        idx_v)
    base = wid * ROWS_PER_W

    def gather(g):
        b = g % NBUF
        return pltpu.async_copy(
            table_hbm.at[idx_v.at[pl.ds(g * CHUNK, CHUNK)]], bufs[b], gsems[b])

    gathers = [None] * N_CHUNKS
    outs = [None] * N_CHUNKS
    for h in range(min(NBUF - 1, N_CHUNKS)):
        gathers[h] = gather(h)
    for g in range(N_CHUNKS):
        b = g % NBUF
        gathers[g].wait()
        outs[g] = pltpu.async_copy(
            bufs[b], out_hbm.at[pl.ds(base + g * CHUNK, CHUNK)], osems[b])
        h = g + NBUF - 1
        if h < N_CHUNKS:
            if h >= NBUF:
                outs[h - NBUF].wait()
            gathers[h] = gather(h)
    for g in range(max(0, N_CHUNKS - NBUF), N_CHUNKS):
        if outs[g] is not None:
            outs[g].wait()


def kernel(tokens, W_E):
    mesh = plsc.VectorSubcoreMesh(core_axis_name="c", subcore_axis_name="s")
    scratch = (
        [pltpu.VMEM((ROWS_PER_W,), jnp.int32)]
        + [pltpu.VMEM((CHUNK, D_MODEL), jnp.float32) for _ in range(NBUF)]
        + [pltpu.SemaphoreType.DMA for _ in range(2 * NBUF)]
    )
    out = pl.kernel(
        _emb_kernel,
        mesh=mesh,
        out_type=jax.ShapeDtypeStruct((N_TOK, D_MODEL), jnp.float32),
        scratch_types=scratch,
    )(W_E, tokens)
    return out.reshape(B, S, D_MODEL)


# speedup vs baseline: 6.3148x; 1.0077x over previous
import jax
import jax.numpy as jnp
from jax import lax
from jax.experimental import pallas as pl
from jax.experimental.pallas import tpu as pltpu
from jax.experimental.pallas import tpu_sc as plsc

B, S = 4, 4096
D_MODEL = 1024
N_TOK = B * S

_info = plsc.get_sparse_core_info()
NC, NS = _info.num_cores, _info.num_subcores
NW = NC * NS
ROWS_PER_W = N_TOK // NW
W_PER_ROW = S // ROWS_PER_W
CHUNK = 16
N_CHUNKS = ROWS_PER_W // CHUNK
NBUF = 7


def _emb_kernel(table_hbm, idx_hbm, out_hbm, idx_v, *rest):
    bufs = rest[:NBUF]
    gsems = rest[NBUF:2 * NBUF]
    osems = rest[2 * NBUF:3 * NBUF]
    wid = lax.axis_index("s") * NC + lax.axis_index("c")
    pltpu.sync_copy(
        idx_hbm.at[wid // W_PER_ROW,
                   pl.ds((wid % W_PER_ROW) * ROWS_PER_W, ROWS_PER_W)],
        idx_v)
    base = wid * ROWS_PER_W

    def gather(g):
        b = g % NBUF
        return pltpu.async_copy(
            table_hbm.at[idx_v.at[pl.ds(g * CHUNK, CHUNK)]], bufs[b], gsems[b])

    gathers = [None] * N_CHUNKS
    outs = [None] * N_CHUNKS
    for h in range(min(NBUF - 1, N_CHUNKS)):
        gathers[h] = gather(h)
    for g in range(N_CHUNKS):
        b = g % NBUF
        gathers[g].wait()
        outs[g] = pltpu.async_copy(
            bufs[b], out_hbm.at[pl.ds(base + g * CHUNK, CHUNK)], osems[b])
        h = g + NBUF - 1
        if h < N_CHUNKS:
            if h >= NBUF:
                outs[h - NBUF].wait()
            gathers[h] = gather(h)
    for g in range(max(0, N_CHUNKS - NBUF), N_CHUNKS):
        if outs[g] is not None:
            outs[g].wait()


def kernel(tokens, W_E):
    mesh = plsc.VectorSubcoreMesh(core_axis_name="c", subcore_axis_name="s")
    scratch = (
        [pltpu.VMEM((ROWS_PER_W,), jnp.int32)]
        + [pltpu.VMEM((CHUNK, D_MODEL), jnp.float32) for _ in range(NBUF)]
        + [pltpu.SemaphoreType.DMA for _ in range(2 * NBUF)]
    )
    out = pl.kernel(
        _emb_kernel,
        mesh=mesh,
        out_type=jax.ShapeDtypeStruct((N_TOK, D_MODEL), jnp.float32),
        scratch_types=scratch,
        compiler_params=pltpu.CompilerParams(skip_device_barrier=True),
    )(W_E, tokens)
    return out.reshape(B, S, D_MODEL)
